# baseline (speedup 1.0000x reference)
import functools

import jax
import jax.numpy as jnp
from jax import lax
from jax.experimental import pallas as pl
from jax.experimental.pallas import tpu as pltpu
from jax.experimental.pallas import tpu_sc as plsc

_C, _T, _H, _W = 3, 64, 512, 512
_ALPHA = 4
_TS = _T // _ALPHA
_NW = 32
_FRAMES_PER_WORKER = _C * _T // _NW


def _copy_body(x_ref, o_ref):
    o_ref[...] = x_ref[...]


_tc_slow_gather = pl.pallas_call(
    _copy_body,
    grid=(_C, _TS),
    in_specs=[
        pl.BlockSpec((1, 1, _H, _W), lambda c, p: (c, (21 * p) // 5, 0, 0))
    ],
    out_specs=pl.BlockSpec((1, 1, _H, _W), lambda c, p: (c, p, 0, 0)),
    out_shape=jax.ShapeDtypeStruct((_C, _TS, _H, _W), jnp.float32),
)


@functools.partial(
    pl.kernel,
    mesh=plsc.VectorSubcoreMesh(core_axis_name="c", subcore_axis_name="s"),
    out_type=jax.ShapeDtypeStruct((_C, _T, _H, _W), jnp.float32),
    scratch_types=[pltpu.SemaphoreType.DMA],
)
def _sc_fast_copy(frames_hbm, out_hbm, sem):
    wid = lax.axis_index("s") * 2 + lax.axis_index("c")
    base = wid * _FRAMES_PER_WORKER
    copies = []
    for i in range(_FRAMES_PER_WORKER):
        f = base + i
        c = f // _T
        t = f % _T
        copies.append(
            pltpu.async_copy(frames_hbm.at[c, t], out_hbm.at[c, t], sem)
        )
    for cp in copies:
        cp.wait()


def kernel(frames):
    fast = _sc_fast_copy(frames)
    slow = _tc_slow_gather(frames)
    return (slow, fast)


# speedup vs baseline: 32.2781x; 32.2781x over previous
import functools

import jax
import jax.numpy as jnp
from jax import lax
from jax.experimental import pallas as pl
from jax.experimental.pallas import tpu as pltpu
from jax.experimental.pallas import tpu_sc as plsc

_C, _T, _H, _W = 3, 64, 512, 512
_ALPHA = 4
_TS = _T // _ALPHA
_NW = 32
_FRAMES_PER_WORKER = _C * _T // _NW


def _copy_body(x_ref, o_ref):
    o_ref[...] = x_ref[...]


_tc_slow_gather = pl.pallas_call(
    _copy_body,
    grid=(_C, _TS),
    in_specs=[
        pl.BlockSpec((1, 1, _H, _W), lambda c, p: (c, (21 * p) // 5, 0, 0))
    ],
    out_specs=pl.BlockSpec((1, 1, _H, _W), lambda c, p: (c, p, 0, 0)),
    out_shape=jax.ShapeDtypeStruct((_C, _TS, _H, _W), jnp.float32),
)


_NBUF = 4
_CROWS = 32
_CHUNKS_PER_FRAME = _H // _CROWS
_NCH = _FRAMES_PER_WORKER * _CHUNKS_PER_FRAME
_NSUPER = _NCH // _NBUF


@functools.partial(
    pl.kernel,
    mesh=plsc.VectorSubcoreMesh(core_axis_name="c", subcore_axis_name="s"),
    out_type=jax.ShapeDtypeStruct((_C, _T, _H, _W), jnp.float32),
    scratch_types=[
        [pltpu.VMEM((_CROWS, _W), jnp.float32)] * _NBUF,
        [pltpu.SemaphoreType.DMA] * _NBUF,
        [pltpu.SemaphoreType.DMA] * _NBUF,
    ],
)
def _sc_fast_copy(frames_hbm, out_hbm, bufs, rsems, wsems):
    wid = lax.axis_index("s") * 2 + lax.axis_index("c")
    base = wid * _NCH

    def _src(g):
        f = g // _CHUNKS_PER_FRAME
        r = (g % _CHUNKS_PER_FRAME) * _CROWS
        return frames_hbm.at[f // _T, f % _T, pl.ds(r, _CROWS), :]

    def _dst(g):
        f = g // _CHUNKS_PER_FRAME
        r = (g % _CHUNKS_PER_FRAME) * _CROWS
        return out_hbm.at[f // _T, f % _T, pl.ds(r, _CROWS), :]

    for j in range(_NBUF):
        pltpu.async_copy(_src(base + j), bufs[j], rsems[j])

    def body(it, carry):
        g0 = base + it * _NBUF
        for j in range(_NBUF):
            pltpu.make_async_copy(_src(g0 + j), bufs[j], rsems[j]).wait()
            pltpu.async_copy(bufs[j], _dst(g0 + j), wsems[j])

        @pl.when(it < _NSUPER - 1)
        def _():
            for j in range(_NBUF):
                pltpu.make_async_copy(bufs[j], _dst(g0 + j), wsems[j]).wait()
                pltpu.async_copy(_src(g0 + _NBUF + j), bufs[j], rsems[j])

        return carry

    lax.fori_loop(0, _NSUPER, body, 0)

    g_last = base + (_NSUPER - 1) * _NBUF
    for j in range(_NBUF):
        pltpu.make_async_copy(bufs[j], _dst(g_last + j), wsems[j]).wait()


def kernel(frames):
    fast = _sc_fast_copy(frames)
    slow = _tc_slow_gather(frames)
    return (slow, fast)
